# Initial kernel scaffold; baseline (speedup 1.0000x reference)
import jax
import jax.numpy as jnp
from jax.experimental import pallas as pl


def kernel(x, emb_weight):
    raise NotImplementedError("write your pallas kernel here")


# speedup vs baseline: 1.1017x; 1.1017x over previous
import functools

import jax
import jax.numpy as jnp
from jax import lax
from jax.experimental import pallas as pl
from jax.experimental.pallas import tpu as pltpu
from jax.experimental.pallas import tpu_sc as plsc


def _make_gather(n_total: int, dim: int, num_workers: int, chunk: int):
  per_w = n_total // num_workers
  n_chunks = per_w // chunk
  mesh = plsc.VectorSubcoreMesh(core_axis_name="c", subcore_axis_name="s")
  nc = mesh.num_cores

  @functools.partial(
      pl.kernel,
      out_type=jax.ShapeDtypeStruct((n_total, dim), jnp.float32),
      mesh=mesh,
      scratch_types=[
          pltpu.VMEM((chunk,), jnp.int32),
          pltpu.VMEM((chunk, dim), jnp.float32),
          pltpu.SemaphoreType.DMA,
      ],
      compiler_params=pltpu.CompilerParams(use_tc_tiling_on_sc=False),
  )
  def gather_kernel(table_hbm, idx_hbm, out_hbm, idx_v, rows_v, sem):
    wid = lax.axis_index("s") * nc + lax.axis_index("c")
    base = wid * per_w

    @pl.loop(0, n_chunks)
    def _chunk_loop(g):
      off = base + g * chunk
      pltpu.sync_copy(idx_hbm.at[pl.ds(off, chunk)], idx_v)
      pltpu.async_copy(table_hbm.at[idx_v], rows_v, sem).wait()
      pltpu.sync_copy(rows_v, out_hbm.at[pl.ds(off, chunk), :])

  return gather_kernel


def kernel(x, emb_weight):
  b, f = x.shape
  n = b * f
  dim = emb_weight.shape[1]
  flat_idx = x.reshape(n)
  out = _make_gather(n, dim, 32, 1024)(emb_weight, flat_idx)
  return out.reshape(b, f, dim)


# speedup vs baseline: 4.3415x; 3.9407x over previous
import functools

import jax
import jax.numpy as jnp
from jax import lax
from jax.experimental import pallas as pl
from jax.experimental.pallas import tpu as pltpu
from jax.experimental.pallas import tpu_sc as plsc


def _make_gather(batch: int, fields: int, dim: int, num_workers: int,
                 rows_per_chunk: int):
  per_w = batch // num_workers
  n_chunks = per_w // rows_per_chunk
  mesh = plsc.VectorSubcoreMesh(core_axis_name="c", subcore_axis_name="s")
  nc = mesh.num_cores

  @functools.partial(
      pl.kernel,
      out_type=jax.ShapeDtypeStruct((batch, fields, dim), jnp.float32),
      mesh=mesh,
      scratch_types=[
          pltpu.VMEM((rows_per_chunk, fields), jnp.int32),
          pltpu.VMEM((rows_per_chunk, fields, dim), jnp.float32),
          pltpu.SemaphoreType.DMA,
      ],
      compiler_params=pltpu.CompilerParams(use_tc_tiling_on_sc=False),
  )
  def gather_kernel(table_hbm, idx_hbm, out_hbm, idx_v, rows_v, sem):
    wid = lax.axis_index("s") * nc + lax.axis_index("c")
    base = wid * per_w

    @pl.loop(0, n_chunks)
    def _chunk_loop(g):
      row0 = base + g * rows_per_chunk
      pltpu.sync_copy(idx_hbm.at[pl.ds(row0, rows_per_chunk), :], idx_v)
      for r in range(rows_per_chunk):
        pltpu.async_copy(table_hbm.at[idx_v.at[r]], rows_v.at[r], sem)
      for r in range(rows_per_chunk):
        pltpu.make_async_copy(table_hbm.at[idx_v.at[r]], rows_v.at[r], sem).wait()
      pltpu.sync_copy(rows_v, out_hbm.at[pl.ds(row0, rows_per_chunk), :, :])

  return gather_kernel


def kernel(x, emb_weight):
  b, f = x.shape
  dim = emb_weight.shape[1]
  return _make_gather(b, f, dim, 32, 16)(emb_weight, x)
